# Initial kernel scaffold; baseline (speedup 1.0000x reference)
import jax
import jax.numpy as jnp
from jax.experimental import pallas as pl


def kernel(x, r_w1, r_b1, r_w2, r_b2, r_w3, r_b3, temperature, pre_w, pre_b, pre_g, pre_beta, mlp_w1, mlp_b1, mlp_w2, mlp_b2, post_w, post_b, post_g, post_beta):
    raise NotImplementedError("write your pallas kernel here")


# speedup vs baseline: 7.2568x; 7.2568x over previous
import functools

import jax
import jax.numpy as jnp
import numpy as np
from jax import lax
from jax.experimental import pallas as pl
from jax.experimental.pallas import tpu as pltpu
from jax.experimental.pallas import tpu_sc as plsc

B, S, D = 1, 2048, 768
NPRE, NMLP, NPOST, K = 16, 16, 16, 8
TOT = NPRE * NMLP * NPOST
HID = [D * (2 + i // 4) for i in range(NMLP)]
HMAX = max(HID)
T = B * S
P = T * K

TM = 256
NPAD = P + 16 * TM
NTILES = NPAD // TM
TH = 768
NH = HMAX // TH

_HIGH = jax.lax.Precision.HIGHEST


def _gelu(z):
    return 0.5 * z * (1.0 + jax.lax.erf(z * np.float32(1.0 / np.sqrt(2.0))))


def _act(z, r):
    a0 = _gelu(z)
    a1 = jnp.maximum(z, 0.0)
    a2 = jnp.tanh(z)
    a3 = z * jax.nn.sigmoid(z)
    return jnp.where(r == 0, a0, jnp.where(r == 1, a1, jnp.where(r == 2, a2, a3)))


def _ln(z, g, b):
    mu = jnp.mean(z, axis=-1, keepdims=True)
    var = jnp.mean((z - mu) * (z - mu), axis=-1, keepdims=True)
    return g * (z - mu) * jax.lax.rsqrt(var + 1e-5) + b


_RT = 128


def _loss_body(s_ref, loss_ref, facc):
    i = pl.program_id(0)
    s = s_ref[...]
    m = jnp.max(s, axis=-1, keepdims=True)
    e = jnp.exp(s - m)
    p0 = e / jnp.sum(e, axis=-1, keepdims=True)
    part = jnp.sum(p0, axis=0, keepdims=True)

    @pl.when(i == 0)
    def _():
        facc[0:1, :] = part

    @pl.when(i > 0)
    def _():
        facc[0:1, :] = facc[0:1, :] + part

    @pl.when(i == (T // _RT) - 1)
    def _():
        freq = facc[0:1, :] / T
        mu = jnp.mean(freq)
        var = jnp.sum((freq - mu) * (freq - mu)) / (TOT - 1)
        loss_ref[...] = jnp.reshape(TOT * var, (1, 1))


def _loss(scores):
    out = pl.pallas_call(
        _loss_body,
        grid=(T // _RT,),
        in_specs=[pl.BlockSpec((_RT, TOT), lambda i: (i, 0))],
        out_specs=pl.BlockSpec((1, 1), lambda i: (0, 0)),
        out_shape=jax.ShapeDtypeStruct((1, 1), jnp.float32),
        scratch_shapes=[pltpu.VMEM((8, TOT), jnp.float32)],
        compiler_params=pltpu.CompilerParams(
            dimension_semantics=("arbitrary",)),
    )(scores)
    return out[0, 0]


def _top8_body(p_ref, tv_ref, ti_ref, pw_ref):
    p = p_ref[...]
    col = jax.lax.broadcasted_iota(jnp.int32, p.shape, 1)
    work = p
    vals = []
    idxs = []
    for _ in range(K):
        m = jnp.max(work, axis=-1, keepdims=True)
        i = jnp.min(jnp.where(work == m, col, TOT), axis=-1, keepdims=True)
        vals.append(m)
        idxs.append(i)
        work = jnp.where(col == i, -jnp.inf, work)
    tv = jnp.concatenate(vals, axis=-1)
    tv_ref[...] = tv
    ti_ref[...] = jnp.concatenate(idxs, axis=-1)
    kth = tv[:, K - 1:K]
    sel = jnp.where(p >= kth, p, 0.0)
    pw_ref[...] = sel / (jnp.sum(sel, axis=-1, keepdims=True) + 1e-8)


def _top8_pw(probs):
    return pl.pallas_call(
        _top8_body,
        grid=(T // _RT,),
        in_specs=[pl.BlockSpec((_RT, TOT), lambda i: (i, 0))],
        out_specs=[
            pl.BlockSpec((_RT, K), lambda i: (i, 0)),
            pl.BlockSpec((_RT, K), lambda i: (i, 0)),
            pl.BlockSpec((_RT, TOT), lambda i: (i, 0)),
        ],
        out_shape=[
            jax.ShapeDtypeStruct((T, K), jnp.float32),
            jax.ShapeDtypeStruct((T, K), jnp.int32),
            jax.ShapeDtypeStruct((T, TOT), jnp.float32),
        ],
        compiler_params=pltpu.CompilerParams(
            dimension_semantics=("arbitrary",)),
    )(probs)


_NW = 32
_CH = 64


def _sc_gather(table, idx):
    L = idx.shape[0]
    per_w = L // _NW
    nch = per_w // _CH
    mesh = plsc.VectorSubcoreMesh(core_axis_name="c", subcore_axis_name="s")

    @functools.partial(
        pl.kernel, mesh=mesh,
        out_type=jax.ShapeDtypeStruct((L, D), jnp.float32),
        scratch_types=[
            pltpu.VMEM((per_w,), jnp.int32),
            pltpu.VMEM((_CH, D), jnp.float32),
            pltpu.VMEM((_CH, D), jnp.float32),
            pltpu.SemaphoreType.DMA,
            pltpu.SemaphoreType.DMA,
            pltpu.SemaphoreType.DMA,
            pltpu.SemaphoreType.DMA,
        ],
    )
    def k(table_hbm, idx_hbm, out_hbm, idx_v, rows0, rows1, g0, g1, s0, s1):
        wid = lax.axis_index("s") * 2 + lax.axis_index("c")
        base = wid * per_w
        pltpu.sync_copy(idx_hbm.at[pl.ds(base, per_w)], idx_v)
        rows = (rows0, rows1)
        gsem = (g0, g1)
        ssem = (s0, s1)

        def gather(c):
            return pltpu.async_copy(
                table_hbm.at[idx_v.at[pl.ds(c * _CH, _CH)]],
                rows[c % 2], gsem[c % 2])

        def put(c):
            return pltpu.async_copy(
                rows[c % 2], out_hbm.at[pl.ds(base + c * _CH, _CH)],
                ssem[c % 2])

        g = [None] * nch
        s = [None] * nch
        g[0] = gather(0)
        if nch > 1:
            g[1] = gather(1)
        for c in range(nch):
            g[c].wait()
            s[c] = put(c)
            if c + 2 < nch:
                s[c].wait()
                g[c + 2] = gather(c + 2)
        for c in range(max(0, nch - 2), nch):
            s[c].wait()

    return k(table, idx)


def _sc_permute(table, src, dst, l_out):
    nch = P // _NW // _CH
    mesh = plsc.VectorSubcoreMesh(core_axis_name="c", subcore_axis_name="s")

    @functools.partial(
        pl.kernel, mesh=mesh,
        out_type=jax.ShapeDtypeStruct((l_out, D), jnp.float32),
        scratch_types=[
            pltpu.VMEM((nch, _CH), jnp.int32),
            pltpu.VMEM((nch, _CH), jnp.int32),
            pltpu.VMEM((_CH, D), jnp.float32),
            pltpu.VMEM((_CH, D), jnp.float32),
            pltpu.SemaphoreType.DMA,
            pltpu.SemaphoreType.DMA,
            pltpu.SemaphoreType.DMA,
            pltpu.SemaphoreType.DMA,
        ],
    )
    def k(table_hbm, src_hbm, dst_hbm, out_hbm, src_v, dst_v, rows0, rows1,
          g0, g1, s0, s1):
        wid = lax.axis_index("s") * 2 + lax.axis_index("c")
        pltpu.sync_copy(src_hbm.at[wid], src_v)
        pltpu.sync_copy(dst_hbm.at[wid], dst_v)
        rows = (rows0, rows1)
        gsem = (g0, g1)
        ssem = (s0, s1)

        def gather(c):
            return pltpu.async_copy(table_hbm.at[src_v.at[c]],
                                    rows[c % 2], gsem[c % 2])

        def scatter(c):
            return pltpu.async_copy(rows[c % 2], out_hbm.at[dst_v.at[c]],
                                    ssem[c % 2])

        g = [None] * nch
        s = [None] * nch
        g[0] = gather(0)
        if nch > 1:
            g[1] = gather(1)
        for c in range(nch):
            g[c].wait()
            s[c] = scatter(c)
            if c + 2 < nch:
                s[c].wait()
                g[c + 2] = gather(c + 2)
        for c in range(max(0, nch - 2), nch):
            s[c].wait()

    return k(table, src, dst)


def _pre_body(te_ref, h_ref, w_ref, b_ref, g_ref, beta_ref, o_ref):
    m = pl.program_id(0)
    e = te_ref[m]
    z = jnp.dot(h_ref[...].astype(jnp.bfloat16), w_ref[0].astype(jnp.bfloat16),
                preferred_element_type=jnp.float32) + b_ref[0]
    z = _ln(z, g_ref[0], beta_ref[0])
    o_ref[...] = _act(z, e % 4)


def _pre_stage(hs, te, w, b, g, beta):
    return pl.pallas_call(
        _pre_body,
        grid_spec=pltpu.PrefetchScalarGridSpec(
            num_scalar_prefetch=1,
            grid=(NTILES,),
            in_specs=[
                pl.BlockSpec((TM, D), lambda m, te: (m, 0)),
                pl.BlockSpec((1, D, D), lambda m, te: (te[m], 0, 0)),
                pl.BlockSpec((1, 1, D), lambda m, te: (te[m], 0, 0)),
                pl.BlockSpec((1, 1, D), lambda m, te: (te[m], 0, 0)),
                pl.BlockSpec((1, 1, D), lambda m, te: (te[m], 0, 0)),
            ],
            out_specs=pl.BlockSpec((TM, D), lambda m, te: (m, 0)),
        ),
        out_shape=jax.ShapeDtypeStruct((NPAD, D), jnp.float32),
        compiler_params=pltpu.CompilerParams(
            dimension_semantics=("arbitrary",)),
    )(te, hs, w.reshape(16, D, D), b.reshape(16, 1, D), g.reshape(16, 1, D),
      beta.reshape(16, 1, D))


def _mlp_body(te_ref, h_ref, w1_ref, b1_ref, w2_ref, b2_ref, o_ref, acc):
    m = pl.program_id(0)
    hch = pl.program_id(1)
    e = te_ref[m]
    nh_e = 2 + e // 4

    @pl.when(hch < nh_e)
    def _():
        z = jnp.dot(h_ref[...].astype(jnp.bfloat16),
                    w1_ref[0].astype(jnp.bfloat16),
                    preferred_element_type=jnp.float32) + b1_ref[0]
        z = _act(z, e % 4)
        part = jnp.dot(z.astype(jnp.bfloat16), w2_ref[0].astype(jnp.bfloat16),
                       preferred_element_type=jnp.float32)

        @pl.when(hch == 0)
        def _():
            acc[...] = part + b2_ref[0]

        @pl.when(hch > 0)
        def _():
            acc[...] = acc[...] + part

    @pl.when(hch == nh_e - 1)
    def _():
        o_ref[...] = acc[...]


def _mlp_stage(hs, te, w1, b1, w2, b2):
    return pl.pallas_call(
        _mlp_body,
        grid_spec=pltpu.PrefetchScalarGridSpec(
            num_scalar_prefetch=1,
            grid=(NTILES, NH),
            in_specs=[
                pl.BlockSpec((TM, D), lambda m, h, te: (m, 0)),
                pl.BlockSpec((1, D, TH),
                             lambda m, h, te: (te[m], 0,
                                               jnp.minimum(h, 1 + te[m] // 4))),
                pl.BlockSpec((1, 1, TH),
                             lambda m, h, te: (te[m], 0,
                                               jnp.minimum(h, 1 + te[m] // 4))),
                pl.BlockSpec((1, TH, D),
                             lambda m, h, te: (te[m],
                                               jnp.minimum(h, 1 + te[m] // 4), 0)),
                pl.BlockSpec((1, 1, D), lambda m, h, te: (te[m], 0, 0)),
            ],
            out_specs=pl.BlockSpec((TM, D), lambda m, h, te: (m, 0)),
            scratch_shapes=[pltpu.VMEM((TM, D), jnp.float32)],
        ),
        out_shape=jax.ShapeDtypeStruct((NPAD, D), jnp.float32),
        compiler_params=pltpu.CompilerParams(
            dimension_semantics=("arbitrary", "arbitrary")),
    )(te, hs, w1, b1.reshape(16, 1, HMAX), w2, b2.reshape(16, 1, D))


def _post_body(te_ref, h_ref, w_ref, b_ref, g_ref, beta_ref, o_ref):
    m = pl.program_id(0)
    e = te_ref[m]
    z = jnp.dot(h_ref[...].astype(jnp.bfloat16), w_ref[0].astype(jnp.bfloat16),
                preferred_element_type=jnp.float32) + b_ref[0]
    o_ref[...] = jnp.where(e % 2 == 0, _ln(z, g_ref[0], beta_ref[0]), z)


def _post_stage(hs, te, w, b, g, beta):
    return pl.pallas_call(
        _post_body,
        grid_spec=pltpu.PrefetchScalarGridSpec(
            num_scalar_prefetch=1,
            grid=(NTILES,),
            in_specs=[
                pl.BlockSpec((TM, D), lambda m, te: (m, 0)),
                pl.BlockSpec((1, D, D), lambda m, te: (te[m], 0, 0)),
                pl.BlockSpec((1, 1, D), lambda m, te: (te[m], 0, 0)),
                pl.BlockSpec((1, 1, D), lambda m, te: (te[m], 0, 0)),
                pl.BlockSpec((1, 1, D), lambda m, te: (te[m], 0, 0)),
            ],
            out_specs=pl.BlockSpec((TM, D), lambda m, te: (m, 0)),
        ),
        out_shape=jax.ShapeDtypeStruct((NPAD, D), jnp.float32),
        compiler_params=pltpu.CompilerParams(
            dimension_semantics=("arbitrary",)),
    )(te, hs, w.reshape(16, D, D), b.reshape(16, 1, D), g.reshape(16, 1, D),
      beta.reshape(16, 1, D))


_TT = 256


def _reduce_body(h3_ref, w_ref, o_ref):
    w = w_ref[...]
    w_eff = jnp.where(w >= 1e-6, w, 0.0)
    o_ref[...] = jnp.sum(w_eff[:, :, None] * h3_ref[...], axis=1)


def _reduce(h3p, top_vals):
    return pl.pallas_call(
        _reduce_body,
        grid=(T // _TT,),
        in_specs=[
            pl.BlockSpec((_TT, K, D), lambda i: (i, 0, 0)),
            pl.BlockSpec((_TT, K), lambda i: (i, 0)),
        ],
        out_specs=pl.BlockSpec((_TT, D), lambda i: (i, 0)),
        out_shape=jax.ShapeDtypeStruct((T, D), jnp.float32),
        compiler_params=pltpu.CompilerParams(
            dimension_semantics=("arbitrary",)),
    )(h3p.reshape(T, K, D), top_vals)


def _layout(key):
    onehot = (key[:, None] == jnp.arange(16, dtype=key.dtype)[None, :])
    cum = jnp.cumsum(onehot.astype(jnp.int32), axis=0)
    rank = jnp.sum(cum * onehot, axis=1) - 1
    counts = cum[-1]
    padded = ((counts + TM - 1) // TM) * TM
    poff = jnp.concatenate([jnp.zeros(1, counts.dtype), jnp.cumsum(padded)[:-1]])
    dst = (jnp.sum(onehot * poff[None, :], axis=1) + rank).astype(jnp.int32)
    ends = poff + padded
    mrow = jnp.arange(NTILES) * TM
    tile_e = jnp.minimum(
        jnp.sum(mrow[:, None] >= ends[None, :], axis=1), 15).astype(jnp.int32)
    return dst, tile_e


@jax.jit
def kernel(x, r_w1, r_b1, r_w2, r_b2, r_w3, r_b3, temperature,
           pre_w, pre_b, pre_g, pre_beta, mlp_w1, mlp_b1, mlp_w2, mlp_b2,
           post_w, post_b, post_g, post_beta):
    x_flat = x.reshape(T, D)

    h = jax.nn.gelu(x @ r_w1 + r_b1, approximate=False)
    h = jax.nn.gelu(h @ r_w2 + r_b2, approximate=False)
    scores = h @ r_w3 + r_b3
    probs = jax.nn.softmax(scores / temperature[0], axis=-1).reshape(T, TOT)
    glbl_loss = _loss(scores.reshape(T, TOT))

    top_vals, top_idx, pathway_weights = _top8_pw(probs)

    p_flat = top_idx.reshape(-1)
    pre_i = p_flat // (NMLP * NPOST)
    rem = p_flat % (NMLP * NPOST)
    mlp_i = rem // NPOST
    post_i = rem % NPOST
    t_flat = jnp.arange(P, dtype=jnp.int32) // K

    d1, te1 = _layout(pre_i)
    d2, te2 = _layout(mlp_i)
    d3, te3 = _layout(post_i)

    nch = P // _NW // _CH
    tfr = t_flat.reshape(_NW, nch, _CH)
    d1r = d1.reshape(_NW, nch, _CH)
    d2r = d2.reshape(_NW, nch, _CH)
    d3r = d3.reshape(_NW, nch, _CH)

    h0s = _sc_permute(x_flat, tfr, d1r, NPAD)
    h1s = _pre_stage(h0s, te1, pre_w, pre_b, pre_g, pre_beta)
    h1g = _sc_permute(h1s, d1r, d2r, NPAD)
    h2s = _mlp_stage(h1g, te2, mlp_w1, mlp_b1, mlp_w2, mlp_b2)
    h2g = _sc_permute(h2s, d2r, d3r, NPAD)
    h3s = _post_stage(h2g, te3, post_w, post_b, post_g, post_beta)
    h3p = _sc_gather(h3s, d3)

    out = _reduce(h3p, top_vals)

    return (out.reshape(B, S, D), glbl_loss,
            pathway_weights.reshape(B, S, TOT))
